# baseline (speedup 1.0000x reference)
import functools

import jax
import jax.numpy as jnp
from jax.experimental import pallas as pl

N = 65536
D_IN = 4
H = 512
Z = 128
K = 1024
BETA = 0.25
NU = 0.1

BLOCK_ROWS = 1024


_INV_SQRT2 = 0.7071067811865476


def _gelu(x):
    return 0.5 * x * (1.0 + jax.lax.erf(x * _INV_SQRT2))


def _vqvae_body(x_ref, mean_ref, std_ref,
                w1_ref, b1_ref, w2_ref, b2_ref, w3_ref, b3_ref,
                cb_ref, asc_ref, abi_ref,
                dw1_ref, db1_ref, dw2_ref, db2_ref, dw3_ref, db3_ref,
                out_ref, loss_ref):
    i = pl.program_id(0)

    cb = cb_ref[...] * asc_ref[...] + abi_ref[...]
    cb_sq = jnp.sum(cb * cb, axis=1)[None, :]
    cb_hi = cb.astype(jnp.bfloat16)
    cb_lo = (cb - cb_hi.astype(jnp.float32)).astype(jnp.bfloat16)

    mean = mean_ref[...]
    std = std_ref[...]

    half = BLOCK_ROWS // 2
    sqs = []
    for s in range(2):
        xn = (x_ref[pl.ds(s * half, half), :] - mean) / std

        h = _gelu(xn @ w1_ref[...] + b1_ref[...])
        h = _gelu(h @ w2_ref[...] + b2_ref[...])
        z = h @ w3_ref[...] + b3_ref[...]

        z_sq = jnp.sum(z * z, axis=1, keepdims=True)
        cross = jax.lax.dot_general(z, cb, (((1,), (1,)), ((), ())))
        d = z_sq - 2.0 * cross + cb_sq

        dmin = jnp.min(d, axis=1, keepdims=True)
        iota = jax.lax.broadcasted_iota(jnp.int32, d.shape, 1)
        idx = jnp.min(jnp.where(d == dmin, iota, K), axis=1)
        onehot = (iota == idx[:, None]).astype(jnp.bfloat16)
        dn = (((1,), (0,)), ((), ()))
        z_q = (jax.lax.dot_general(onehot, cb_hi, dn,
                                   preferred_element_type=jnp.float32)
               + jax.lax.dot_general(onehot, cb_lo, dn,
                                     preferred_element_type=jnp.float32))

        diff = z - z_q
        sqs.append(jnp.sum(diff * diff))

        z_q_st = z + (z_q - z)
        g = _gelu(z_q_st @ dw1_ref[...] + db1_ref[...])
        g = _gelu(g @ dw2_ref[...] + db2_ref[...])
        xr = g @ dw3_ref[...] + db3_ref[...]
        out_ref[pl.ds(s * half, half), :] = xr * std + mean

    sq = (sqs[0] + sqs[1]).reshape(1, 1)

    @pl.when(i == 0)
    def _init():
        loss_ref[...] = sq

    @pl.when(i > 0)
    def _acc():
        loss_ref[...] += sq


@jax.jit
def kernel(x, mean, std, enc_w1, enc_b1, enc_w2, enc_b2, enc_w3, enc_b3,
           codebook, affine_scale, affine_bias,
           dec_w1, dec_b1, dec_w2, dec_b2, dec_w3, dec_b3):
    n = x.shape[0]
    grid = (n // BLOCK_ROWS,)

    b1 = enc_b1.reshape(1, H)
    b2 = enc_b2.reshape(1, H)
    b3 = enc_b3.reshape(1, Z)
    db1 = dec_b1.reshape(1, H)
    db2 = dec_b2.reshape(1, H)
    db3 = dec_b3.reshape(1, D_IN)

    def fixed(shape):
        return pl.BlockSpec(shape, lambda i: (0,) * len(shape))

    out, loss_sum = pl.pallas_call(
        _vqvae_body,
        grid=grid,
        in_specs=[
            pl.BlockSpec((BLOCK_ROWS, D_IN), lambda i: (i, 0)),
            fixed((1, D_IN)), fixed((1, D_IN)),
            fixed((D_IN, H)), fixed((1, H)),
            fixed((H, H)), fixed((1, H)),
            fixed((H, Z)), fixed((1, Z)),
            fixed((K, Z)), fixed((1, Z)), fixed((1, Z)),
            fixed((Z, H)), fixed((1, H)),
            fixed((H, H)), fixed((1, H)),
            fixed((H, D_IN)), fixed((1, D_IN)),
        ],
        out_specs=[
            pl.BlockSpec((BLOCK_ROWS, D_IN), lambda i: (i, 0)),
            pl.BlockSpec((1, 1), lambda i: (0, 0)),
        ],
        out_shape=[
            jax.ShapeDtypeStruct((n, D_IN), jnp.float32),
            jax.ShapeDtypeStruct((1, 1), jnp.float32),
        ],
    )(x, mean, std, enc_w1, b1, enc_w2, b2, enc_w3, b3,
      codebook, affine_scale, affine_bias,
      dec_w1, db1, dec_w2, db2, dec_w3, db3)

    m = loss_sum[0, 0] / (n * Z)
    vq_loss = (1.0 - BETA) * m + BETA * m
    return (out, vq_loss)


# speedup vs baseline: 1.0472x; 1.0472x over previous
import functools

import jax
import jax.numpy as jnp
from jax.experimental import pallas as pl

N = 65536
D_IN = 4
H = 512
Z = 128
K = 1024
BETA = 0.25
NU = 0.1

BLOCK_ROWS = 1024


_INV_SQRT2 = 0.7071067811865476


def _gelu(x):
    return 0.5 * x * (1.0 + jax.lax.erf(x * _INV_SQRT2))


def _vqvae_body(x_ref, mean_ref, std_ref,
                w1_ref, b1_ref, w2_ref, b2_ref, w3_ref, b3_ref,
                cb_ref, asc_ref, abi_ref,
                dw1_ref, db1_ref, dw2_ref, db2_ref, dw3_ref, db3_ref,
                out_ref, loss_ref):
    i = pl.program_id(0)

    cb = cb_ref[...] * asc_ref[...] + abi_ref[...]
    cb_sq = jnp.sum(cb * cb, axis=1)[None, :]
    cb_hi = cb.astype(jnp.bfloat16)
    cb_lo = (cb - cb_hi.astype(jnp.float32)).astype(jnp.bfloat16)

    xn = (x_ref[...] - mean_ref[...]) / std_ref[...]

    h = _gelu(xn @ w1_ref[...] + b1_ref[...])
    h = _gelu(h @ w2_ref[...] + b2_ref[...])
    z = h @ w3_ref[...] + b3_ref[...]

    z_sq = jnp.sum(z * z, axis=1, keepdims=True)
    cross = jax.lax.dot_general(z, cb, (((1,), (1,)), ((), ())))
    d = z_sq - 2.0 * cross + cb_sq

    dmin = jnp.min(d, axis=1, keepdims=True)
    iota = jax.lax.broadcasted_iota(jnp.int32, d.shape, 1)
    idx = jnp.min(jnp.where(d == dmin, iota, K), axis=1)
    onehot = (iota == idx[:, None]).astype(jnp.bfloat16)
    dn = (((1,), (0,)), ((), ()))
    z_q = (jax.lax.dot_general(onehot, cb_hi, dn,
                               preferred_element_type=jnp.float32)
           + jax.lax.dot_general(onehot, cb_lo, dn,
                                 preferred_element_type=jnp.float32))

    diff = z - z_q
    sq = jnp.sum(diff * diff).reshape(1, 1)

    z_q_st = z + (z_q - z)
    g = _gelu(z_q_st @ dw1_ref[...] + db1_ref[...])
    g = _gelu(g @ dw2_ref[...] + db2_ref[...])
    xr = g @ dw3_ref[...] + db3_ref[...]
    out_ref[...] = xr * std_ref[...] + mean_ref[...]

    @pl.when(i == 0)
    def _init():
        loss_ref[...] = sq

    @pl.when(i > 0)
    def _acc():
        loss_ref[...] += sq


@jax.jit
def kernel(x, mean, std, enc_w1, enc_b1, enc_w2, enc_b2, enc_w3, enc_b3,
           codebook, affine_scale, affine_bias,
           dec_w1, dec_b1, dec_w2, dec_b2, dec_w3, dec_b3):
    n = x.shape[0]
    grid = (n // BLOCK_ROWS,)

    b1 = enc_b1.reshape(1, H)
    b2 = enc_b2.reshape(1, H)
    b3 = enc_b3.reshape(1, Z)
    db1 = dec_b1.reshape(1, H)
    db2 = dec_b2.reshape(1, H)
    db3 = dec_b3.reshape(1, D_IN)

    def fixed(shape):
        return pl.BlockSpec(shape, lambda i: (0,) * len(shape))

    out, loss_sum = pl.pallas_call(
        _vqvae_body,
        grid=grid,
        in_specs=[
            pl.BlockSpec((BLOCK_ROWS, D_IN), lambda i: (i, 0)),
            fixed((1, D_IN)), fixed((1, D_IN)),
            fixed((D_IN, H)), fixed((1, H)),
            fixed((H, H)), fixed((1, H)),
            fixed((H, Z)), fixed((1, Z)),
            fixed((K, Z)), fixed((1, Z)), fixed((1, Z)),
            fixed((Z, H)), fixed((1, H)),
            fixed((H, H)), fixed((1, H)),
            fixed((H, D_IN)), fixed((1, D_IN)),
        ],
        out_specs=[
            pl.BlockSpec((BLOCK_ROWS, D_IN), lambda i: (i, 0)),
            pl.BlockSpec((1, 1), lambda i: (0, 0)),
        ],
        out_shape=[
            jax.ShapeDtypeStruct((n, D_IN), jnp.float32),
            jax.ShapeDtypeStruct((1, 1), jnp.float32),
        ],
    )(x, mean, std, enc_w1, b1, enc_w2, b2, enc_w3, b3,
      codebook, affine_scale, affine_bias,
      dec_w1, db1, dec_w2, db2, dec_w3, db3)

    m = loss_sum[0, 0] / (n * Z)
    vq_loss = (1.0 - BETA) * m + BETA * m
    return (out, vq_loss)


# speedup vs baseline: 1.0941x; 1.0447x over previous
import functools

import jax
import jax.numpy as jnp
from jax.experimental import pallas as pl

N = 65536
D_IN = 4
H = 512
Z = 128
K = 1024
BETA = 0.25
NU = 0.1

BLOCK_ROWS = 2048


_INV_SQRT2 = 0.7071067811865476


def _gelu(x):
    return 0.5 * x * (1.0 + jax.lax.erf(x * _INV_SQRT2))


def _vqvae_body(x_ref, mean_ref, std_ref,
                w1_ref, b1_ref, w2_ref, b2_ref, w3_ref, b3_ref,
                cb_ref, asc_ref, abi_ref,
                dw1_ref, db1_ref, dw2_ref, db2_ref, dw3_ref, db3_ref,
                out_ref, loss_ref):
    i = pl.program_id(0)

    cb = cb_ref[...] * asc_ref[...] + abi_ref[...]
    cb_sq = jnp.sum(cb * cb, axis=1)[None, :]
    cb_hi = cb.astype(jnp.bfloat16)
    cb_lo = (cb - cb_hi.astype(jnp.float32)).astype(jnp.bfloat16)

    mean = mean_ref[...]
    std = std_ref[...]

    half = BLOCK_ROWS // 2
    sqs = []
    for s in range(2):
        xn = (x_ref[pl.ds(s * half, half), :] - mean) / std

        h = _gelu(xn @ w1_ref[...] + b1_ref[...])
        h = _gelu(h @ w2_ref[...] + b2_ref[...])
        z = h @ w3_ref[...] + b3_ref[...]

        z_sq = jnp.sum(z * z, axis=1, keepdims=True)
        cross = jax.lax.dot_general(z, cb, (((1,), (1,)), ((), ())))
        d = z_sq - 2.0 * cross + cb_sq

        dmin = jnp.min(d, axis=1, keepdims=True)
        iota = jax.lax.broadcasted_iota(jnp.int32, d.shape, 1)
        idx = jnp.min(jnp.where(d == dmin, iota, K), axis=1)
        onehot = (iota == idx[:, None]).astype(jnp.bfloat16)
        dn = (((1,), (0,)), ((), ()))
        z_q = (jax.lax.dot_general(onehot, cb_hi, dn,
                                   preferred_element_type=jnp.float32)
               + jax.lax.dot_general(onehot, cb_lo, dn,
                                     preferred_element_type=jnp.float32))

        diff = z - z_q
        sqs.append(jnp.sum(diff * diff))

        z_q_st = z + (z_q - z)
        g = _gelu(z_q_st @ dw1_ref[...] + db1_ref[...])
        g = _gelu(g @ dw2_ref[...] + db2_ref[...])
        xr = g @ dw3_ref[...] + db3_ref[...]
        out_ref[pl.ds(s * half, half), :] = xr * std + mean

    sq = (sqs[0] + sqs[1]).reshape(1, 1)

    @pl.when(i == 0)
    def _init():
        loss_ref[...] = sq

    @pl.when(i > 0)
    def _acc():
        loss_ref[...] += sq


@jax.jit
def kernel(x, mean, std, enc_w1, enc_b1, enc_w2, enc_b2, enc_w3, enc_b3,
           codebook, affine_scale, affine_bias,
           dec_w1, dec_b1, dec_w2, dec_b2, dec_w3, dec_b3):
    n = x.shape[0]
    grid = (n // BLOCK_ROWS,)

    b1 = enc_b1.reshape(1, H)
    b2 = enc_b2.reshape(1, H)
    b3 = enc_b3.reshape(1, Z)
    db1 = dec_b1.reshape(1, H)
    db2 = dec_b2.reshape(1, H)
    db3 = dec_b3.reshape(1, D_IN)

    def fixed(shape):
        return pl.BlockSpec(shape, lambda i: (0,) * len(shape))

    out, loss_sum = pl.pallas_call(
        _vqvae_body,
        grid=grid,
        in_specs=[
            pl.BlockSpec((BLOCK_ROWS, D_IN), lambda i: (i, 0)),
            fixed((1, D_IN)), fixed((1, D_IN)),
            fixed((D_IN, H)), fixed((1, H)),
            fixed((H, H)), fixed((1, H)),
            fixed((H, Z)), fixed((1, Z)),
            fixed((K, Z)), fixed((1, Z)), fixed((1, Z)),
            fixed((Z, H)), fixed((1, H)),
            fixed((H, H)), fixed((1, H)),
            fixed((H, D_IN)), fixed((1, D_IN)),
        ],
        out_specs=[
            pl.BlockSpec((BLOCK_ROWS, D_IN), lambda i: (i, 0)),
            pl.BlockSpec((1, 1), lambda i: (0, 0)),
        ],
        out_shape=[
            jax.ShapeDtypeStruct((n, D_IN), jnp.float32),
            jax.ShapeDtypeStruct((1, 1), jnp.float32),
        ],
    )(x, mean, std, enc_w1, b1, enc_w2, b2, enc_w3, b3,
      codebook, affine_scale, affine_bias,
      dec_w1, db1, dec_w2, db2, dec_w3, db3)

    m = loss_sum[0, 0] / (n * Z)
    vq_loss = (1.0 - BETA) * m + BETA * m
    return (out, vq_loss)


# speedup vs baseline: 1.1175x; 1.0214x over previous
import functools

import jax
import jax.numpy as jnp
from jax.experimental import pallas as pl

N = 65536
D_IN = 4
H = 512
Z = 128
K = 1024
BETA = 0.25
NU = 0.1

BLOCK_ROWS = 4096
CHAIN_ROWS = 1024


_INV_SQRT2 = 0.7071067811865476


def _gelu(x):
    return 0.5 * x * (1.0 + jax.lax.erf(x * _INV_SQRT2))


def _vqvae_body(x_ref, mean_ref, std_ref,
                w1_ref, b1_ref, w2_ref, b2_ref, w3_ref, b3_ref,
                cb_ref, asc_ref, abi_ref,
                dw1_ref, db1_ref, dw2_ref, db2_ref, dw3_ref, db3_ref,
                out_ref, loss_ref):
    i = pl.program_id(0)

    cb = cb_ref[...] * asc_ref[...] + abi_ref[...]
    cb_sq = jnp.sum(cb * cb, axis=1)[None, :]
    cb_hi = cb.astype(jnp.bfloat16)
    cb_lo = (cb - cb_hi.astype(jnp.float32)).astype(jnp.bfloat16)

    mean = mean_ref[...]
    std = std_ref[...]

    half = CHAIN_ROWS
    sqs = []
    for s in range(BLOCK_ROWS // CHAIN_ROWS):
        xn = (x_ref[pl.ds(s * half, half), :] - mean) / std

        h = _gelu(xn @ w1_ref[...] + b1_ref[...])
        h = _gelu(h @ w2_ref[...] + b2_ref[...])
        z = h @ w3_ref[...] + b3_ref[...]

        z_sq = jnp.sum(z * z, axis=1, keepdims=True)
        cross = jax.lax.dot_general(z, cb, (((1,), (1,)), ((), ())))
        d = z_sq - 2.0 * cross + cb_sq

        dmin = jnp.min(d, axis=1, keepdims=True)
        iota = jax.lax.broadcasted_iota(jnp.int32, d.shape, 1)
        idx = jnp.min(jnp.where(d == dmin, iota, K), axis=1)
        onehot = (iota == idx[:, None]).astype(jnp.bfloat16)
        dn = (((1,), (0,)), ((), ()))
        z_q = (jax.lax.dot_general(onehot, cb_hi, dn,
                                   preferred_element_type=jnp.float32)
               + jax.lax.dot_general(onehot, cb_lo, dn,
                                     preferred_element_type=jnp.float32))

        diff = z - z_q
        sqs.append(jnp.sum(diff * diff))

        z_q_st = z + (z_q - z)
        g = _gelu(z_q_st @ dw1_ref[...] + db1_ref[...])
        g = _gelu(g @ dw2_ref[...] + db2_ref[...])
        xr = g @ dw3_ref[...] + db3_ref[...]
        out_ref[pl.ds(s * half, half), :] = xr * std + mean

    sq = sum(sqs[1:], sqs[0]).reshape(1, 1)

    @pl.when(i == 0)
    def _init():
        loss_ref[...] = sq

    @pl.when(i > 0)
    def _acc():
        loss_ref[...] += sq


@jax.jit
def kernel(x, mean, std, enc_w1, enc_b1, enc_w2, enc_b2, enc_w3, enc_b3,
           codebook, affine_scale, affine_bias,
           dec_w1, dec_b1, dec_w2, dec_b2, dec_w3, dec_b3):
    n = x.shape[0]
    grid = (n // BLOCK_ROWS,)

    b1 = enc_b1.reshape(1, H)
    b2 = enc_b2.reshape(1, H)
    b3 = enc_b3.reshape(1, Z)
    db1 = dec_b1.reshape(1, H)
    db2 = dec_b2.reshape(1, H)
    db3 = dec_b3.reshape(1, D_IN)

    def fixed(shape):
        return pl.BlockSpec(shape, lambda i: (0,) * len(shape))

    out, loss_sum = pl.pallas_call(
        _vqvae_body,
        grid=grid,
        in_specs=[
            pl.BlockSpec((BLOCK_ROWS, D_IN), lambda i: (i, 0)),
            fixed((1, D_IN)), fixed((1, D_IN)),
            fixed((D_IN, H)), fixed((1, H)),
            fixed((H, H)), fixed((1, H)),
            fixed((H, Z)), fixed((1, Z)),
            fixed((K, Z)), fixed((1, Z)), fixed((1, Z)),
            fixed((Z, H)), fixed((1, H)),
            fixed((H, H)), fixed((1, H)),
            fixed((H, D_IN)), fixed((1, D_IN)),
        ],
        out_specs=[
            pl.BlockSpec((BLOCK_ROWS, D_IN), lambda i: (i, 0)),
            pl.BlockSpec((1, 1), lambda i: (0, 0)),
        ],
        out_shape=[
            jax.ShapeDtypeStruct((n, D_IN), jnp.float32),
            jax.ShapeDtypeStruct((1, 1), jnp.float32),
        ],
    )(x, mean, std, enc_w1, b1, enc_w2, b2, enc_w3, b3,
      codebook, affine_scale, affine_bias,
      dec_w1, db1, dec_w2, db2, dec_w3, db3)

    m = loss_sum[0, 0] / (n * Z)
    vq_loss = (1.0 - BETA) * m + BETA * m
    return (out, vq_loss)
